# Initial kernel scaffold; baseline (speedup 1.0000x reference)
import jax
import jax.numpy as jnp
from jax.experimental import pallas as pl


def kernel(inputs):
    raise NotImplementedError("write your pallas kernel here")


# speedup vs baseline: 1.2155x; 1.2155x over previous
import functools

import jax
import jax.numpy as jnp
from jax import lax
from jax.experimental import pallas as pl
from jax.experimental.pallas import tpu as pltpu
from jax.experimental.pallas import tpu_sc as plsc

B, H, W, C = 8, 224, 224, 96
NBINS = 256
NB_INT = NBINS + 1
HIST_W = NB_INT * C
PIX = H * W
TILES_PER_B = 4
PIX_PER_TILE = PIX // TILES_PER_B
CHUNK_PIX = 224
CHUNK = CHUNK_PIX * C
NCHUNKS = PIX_PER_TILE // CHUNK_PIX
OUT_ROWS = NBINS // TILES_PER_B
OUT_W = OUT_ROWS * C
LANES = 16
GROUPS = C // LANES


def _body(x_hbm, out_hbm, hist_v, buf0, buf1, den_v, den4_v,
          sh_hist, sh_den, sem0, sem1):
    cid = lax.axis_index("c")
    sid = lax.axis_index("s")
    b = cid * TILES_PER_B + sid // TILES_PER_B
    q = sid % TILES_PER_B
    tile_base = b * (PIX * C) + q * (PIX_PER_TILE * C)

    zeros = jnp.zeros((LANES,), jnp.float32)
    ones = jnp.ones((LANES,), jnp.float32)
    iota = lax.iota(jnp.int32, LANES)
    cvecs = [iota + j * LANES for j in range(GROUPS)]

    def zero_body(i, _):
        hist_v[pl.ds(i * LANES, LANES)] = zeros
        return 0

    lax.fori_loop(0, HIST_W // LANES, zero_body, 0)

    def chunk_compute(buf):
        def pix_body(it, _):
            base = it * C
            for j in range(GROUPS):
                v = buf[pl.ds(base + j * LANES, LANES)]
                y = jnp.minimum(v * jnp.float32(NB_INT), jnp.float32(256.5))
                idx = y.astype(jnp.int32) * C + cvecs[j]
                plsc.addupdate_scatter(hist_v, [idx], ones)
            return 0

        lax.fori_loop(0, CHUNK_PIX, pix_body, 0)

    def start(n, buf, sem):
        pltpu.async_copy(x_hbm.at[pl.ds(tile_base + n * CHUNK, CHUNK)], buf, sem)

    def wait(buf, sem):
        pltpu.make_async_copy(x_hbm.at[pl.ds(0, CHUNK)], buf, sem).wait()

    start(0, buf0, sem0)

    def pair_body(k, _):
        start(2 * k + 1, buf1, sem1)
        wait(buf0, sem0)
        chunk_compute(buf0)

        @pl.when(k < NCHUNKS // 2 - 1)
        def _():
            start(2 * k + 2, buf0, sem0)

        wait(buf1, sem1)
        chunk_compute(buf1)
        return 0

    lax.fori_loop(0, NCHUNKS // 2, pair_body, 0)

    pltpu.sync_copy(hist_v, sh_hist.at[pl.ds(sid * HIST_W, HIST_W)])
    plsc.subcore_barrier()

    group = (sid // TILES_PER_B) * TILES_PER_B
    w0 = q * OUT_W + C
    pltpu.sync_copy(sh_hist.at[pl.ds(group * HIST_W + w0, OUT_W)],
                    buf0.at[pl.ds(0, OUT_W)])
    for j in range(1, TILES_PER_B):
        pltpu.sync_copy(sh_hist.at[pl.ds((group + j) * HIST_W + w0, OUT_W)],
                        buf1.at[pl.ds(0, OUT_W)])

        def add_body(i, _):
            o = i * LANES
            buf0[pl.ds(o, LANES)] = buf0[pl.ds(o, LANES)] + buf1[pl.ds(o, LANES)]
            return 0

        lax.fori_loop(0, OUT_W // LANES, add_body, 0)

    def den_body(r, acc):
        base = r * C
        return tuple(acc[j] + buf0[pl.ds(base + j * LANES, LANES)]
                     for j in range(GROUPS))

    acc = lax.fori_loop(0, OUT_ROWS, den_body, tuple([zeros] * GROUPS))
    for j in range(GROUPS):
        den_v[pl.ds(j * LANES, LANES)] = acc[j]
    pltpu.sync_copy(den_v, sh_den.at[pl.ds(sid * C, C)])
    plsc.subcore_barrier()

    pltpu.sync_copy(sh_den.at[pl.ds(group * C, TILES_PER_B * C)], den4_v)
    invs = []
    for j in range(GROUPS):
        d = den4_v[pl.ds(j * LANES, LANES)]
        for r in range(1, TILES_PER_B):
            d = d + den4_v[pl.ds(r * C + j * LANES, LANES)]
        d = jnp.maximum(d, jnp.float32(1e-7))
        invs.append(jnp.float32(1.0) / d)

    def out_body(r, _):
        base = r * C
        for j in range(GROUPS):
            o = base + j * LANES
            buf1[pl.ds(o, LANES)] = buf0[pl.ds(o, LANES)] * invs[j]
        return 0

    lax.fori_loop(0, OUT_ROWS, out_body, 0)
    pltpu.sync_copy(buf1.at[pl.ds(0, OUT_W)],
                    out_hbm.at[pl.ds(b * (NBINS * C) + q * OUT_W, OUT_W)])


@jax.jit
def _hist(x_flat):
    mesh = plsc.VectorSubcoreMesh(core_axis_name="c", subcore_axis_name="s")
    run = pl.kernel(
        _body,
        out_type=jax.ShapeDtypeStruct((B * NBINS * C,), jnp.float32),
        mesh=mesh,
        compiler_params=pltpu.CompilerParams(needs_layout_passes=False),
        scratch_types=[
            pltpu.VMEM((HIST_W,), jnp.float32),
            pltpu.VMEM((CHUNK,), jnp.float32),
            pltpu.VMEM((CHUNK,), jnp.float32),
            pltpu.VMEM((C,), jnp.float32),
            pltpu.VMEM((TILES_PER_B * C,), jnp.float32),
            pltpu.VMEM_SHARED((16 * HIST_W,), jnp.float32),
            pltpu.VMEM_SHARED((16 * C,), jnp.float32),
            pltpu.SemaphoreType.DMA,
            pltpu.SemaphoreType.DMA,
        ],
    )
    return run(x_flat)


def kernel(inputs):
    out = _hist(inputs.reshape(-1))
    return out.reshape(B, NBINS, C)


# speedup vs baseline: 6.8278x; 5.6173x over previous
import jax
import jax.numpy as jnp
from jax import lax
from jax.experimental import pallas as pl
from jax.experimental.pallas import tpu as pltpu
from jax.experimental.pallas import tpu_sc as plsc

B, H, W, C = 8, 224, 224, 96
NBINS = 256
NB_INT = NBINS + 1
HIST_W = NB_INT * C
TILES_PER_B = 4
ROWS_PER_TILE = H // TILES_PER_B
OUT_ROWS = NBINS // TILES_PER_B
OUT_W = OUT_ROWS * C
LANES = 16
GROUPS = C // LANES


def _body(x_hbm, out_hbm, hist_v, buf0, buf1, den_v, den4_v, outb_v,
          sh_hist, sh_den, sem0, sem1):
    cid = lax.axis_index("c")
    sid = lax.axis_index("s")
    b = cid * TILES_PER_B + sid // TILES_PER_B
    q = sid % TILES_PER_B
    h_base = q * ROWS_PER_TILE

    zeros = jnp.zeros((LANES,), jnp.float32)
    ones = jnp.ones((LANES,), jnp.float32)
    iota = lax.iota(jnp.int32, LANES)
    cvecs = [iota + j * LANES for j in range(GROUPS)]

    @plsc.parallel_loop(0, HIST_W // LANES)
    def _(i):
        hist_v[pl.ds(i * LANES, LANES)] = zeros

    def chunk_compute(buf):
        @plsc.parallel_loop(0, W, unroll=2)
        def _(p):
            for j in range(GROUPS):
                v = buf[p, pl.ds(j * LANES, LANES)]
                y = jnp.minimum(v * jnp.float32(NB_INT), jnp.float32(256.5))
                idx = y.astype(jnp.int32) * C + cvecs[j]
                plsc.addupdate_scatter(hist_v, [idx], ones)

    def start(h, buf, sem):
        pltpu.async_copy(x_hbm.at[b, h_base + h], buf, sem)

    def wait(buf, sem):
        pltpu.make_async_copy(x_hbm.at[0, 0], buf, sem).wait()

    start(0, buf0, sem0)

    def pair_body(k, _):
        start(2 * k + 1, buf1, sem1)
        wait(buf0, sem0)
        chunk_compute(buf0)

        @pl.when(k < ROWS_PER_TILE // 2 - 1)
        def _():
            start(2 * k + 2, buf0, sem0)

        wait(buf1, sem1)
        chunk_compute(buf1)
        return 0

    lax.fori_loop(0, ROWS_PER_TILE // 2, pair_body, 0)

    pltpu.sync_copy(hist_v, sh_hist.at[pl.ds(sid * HIST_W, HIST_W)])
    plsc.subcore_barrier()

    group = (sid // TILES_PER_B) * TILES_PER_B
    w0 = q * OUT_W + C
    pltpu.sync_copy(sh_hist.at[pl.ds(group * HIST_W + w0, OUT_W)],
                    hist_v.at[pl.ds(0, OUT_W)])
    for j in range(1, TILES_PER_B):
        pltpu.sync_copy(sh_hist.at[pl.ds((group + j) * HIST_W + w0, OUT_W)],
                        hist_v.at[pl.ds(OUT_W, OUT_W)])

        @plsc.parallel_loop(0, OUT_W // LANES)
        def _(i):
            o = i * LANES
            hist_v[pl.ds(o, LANES)] = (hist_v[pl.ds(o, LANES)]
                                       + hist_v[pl.ds(OUT_W + o, LANES)])

    @plsc.parallel_loop(0, OUT_ROWS, carry=tuple([zeros] * GROUPS))
    def acc(r, a):
        base = r * C
        return tuple(a[j] + hist_v[pl.ds(base + j * LANES, LANES)]
                     for j in range(GROUPS))

    for j in range(GROUPS):
        den_v[pl.ds(j * LANES, LANES)] = acc[j]
    pltpu.sync_copy(den_v, sh_den.at[pl.ds(sid * C, C)])
    plsc.subcore_barrier()

    pltpu.sync_copy(sh_den.at[pl.ds(group * C, TILES_PER_B * C)], den4_v)
    invs = []
    for j in range(GROUPS):
        d = den4_v[pl.ds(j * LANES, LANES)]
        for r in range(1, TILES_PER_B):
            d = d + den4_v[pl.ds(r * C + j * LANES, LANES)]
        d = jnp.maximum(d, jnp.float32(1e-7))
        invs.append(jnp.float32(1.0) / d)

    @plsc.parallel_loop(0, OUT_ROWS)
    def _(r):
        base = r * C
        for j in range(GROUPS):
            outb_v[r, pl.ds(j * LANES, LANES)] = (
                hist_v[pl.ds(base + j * LANES, LANES)] * invs[j])

    pltpu.sync_copy(outb_v, out_hbm.at[b, pl.ds(q * OUT_ROWS, OUT_ROWS)])


@jax.jit
def kernel(inputs):
    mesh = plsc.VectorSubcoreMesh(core_axis_name="c", subcore_axis_name="s")
    run = pl.kernel(
        _body,
        out_type=jax.ShapeDtypeStruct((B, NBINS, C), jnp.float32),
        mesh=mesh,
        compiler_params=pltpu.CompilerParams(needs_layout_passes=False),
        scratch_types=[
            pltpu.VMEM((HIST_W,), jnp.float32),
            pltpu.VMEM((W, C), jnp.float32),
            pltpu.VMEM((W, C), jnp.float32),
            pltpu.VMEM((C,), jnp.float32),
            pltpu.VMEM((TILES_PER_B * C,), jnp.float32),
            pltpu.VMEM((OUT_ROWS, C), jnp.float32),
            pltpu.VMEM_SHARED((16 * HIST_W,), jnp.float32),
            pltpu.VMEM_SHARED((16 * C,), jnp.float32),
            pltpu.SemaphoreType.DMA,
            pltpu.SemaphoreType.DMA,
        ],
    )
    return run(inputs)


# speedup vs baseline: 11.0386x; 1.6167x over previous
import jax
import jax.numpy as jnp
from jax import lax
from jax.experimental import pallas as pl
from jax.experimental.pallas import tpu as pltpu
from jax.experimental.pallas import tpu_sc as plsc

B, H, W, C = 8, 224, 224, 96
NBINS = 256
NB_INT = NBINS + 1
TILES_PER_B = 4
C_PER_TILE = C // TILES_PER_B
CPASS = 8
NPASS = C_PER_TILE // CPASS
HC = 16
NCHUNK = H // HC
LANES = 16
WK = W // LANES
HROW = 273
BIN0 = 15
CROW = LANES * HROW
H16 = CPASS * CROW
KF = NBINS // LANES


def _body(x_hbm, out_hbm, h16_v, buf0, buf1, outb_v, sem0, sem1):
    cid = lax.axis_index("c")
    sid = lax.axis_index("s")
    b = cid * TILES_PER_B + sid // TILES_PER_B
    q = sid % TILES_PER_B
    c0 = q * C_PER_TILE

    zeros = jnp.zeros((LANES,), jnp.float32)
    ones = jnp.ones((LANES,), jnp.float32)
    iota = lax.iota(jnp.int32, LANES)
    lbase = iota * HROW + BIN0

    @plsc.parallel_loop(0, H16 // LANES)
    def _(i):
        h16_v[pl.ds(i * LANES, LANES)] = zeros

    def chunk_compute(buf):
        @plsc.parallel_loop(0, CPASS * HC)
        def _(i):
            ci = i >> 4
            h = i & 15
            cb = lbase + ci * CROW
            for k in range(WK):
                v = buf[h, ci, pl.ds(k * LANES, LANES)]
                y = jnp.minimum(v * jnp.float32(NB_INT), jnp.float32(256.5))
                idx = y.astype(jnp.int32) + cb
                plsc.addupdate_scatter(h16_v, [idx], ones)

    def start(p, n, buf, sem):
        pltpu.async_copy(
            x_hbm.at[b, pl.ds(n * HC, HC), pl.ds(c0 + p * CPASS, CPASS), :],
            buf, sem)

    def wait(buf, sem):
        pltpu.make_async_copy(
            x_hbm.at[0, pl.ds(0, HC), pl.ds(0, CPASS), :], buf, sem).wait()

    def pass_body(p, _):
        start(p, 0, buf0, sem0)

        def pair_body(k, _):
            start(p, 2 * k + 1, buf1, sem1)
            wait(buf0, sem0)
            chunk_compute(buf0)

            @pl.when(k < NCHUNK // 2 - 1)
            def _():
                start(p, 2 * k + 2, buf0, sem0)

            wait(buf1, sem1)
            chunk_compute(buf1)
            return 0

        lax.fori_loop(0, NCHUNK // 2, pair_body, 0)

        def fold_body(ci, _):
            base = ci * CROW

            @plsc.parallel_loop(0, LANES, carry=tuple([zeros] * (KF + 1)))
            def acc(lane, a):
                row = base + lane * HROW
                nxt = []
                for k in range(KF + 1):
                    o = row + k * LANES
                    nxt.append(a[k] + h16_v[pl.ds(o, LANES)])
                    h16_v[pl.ds(o, LANES)] = zeros
                return tuple(nxt)

            dvec = acc[1]
            for k in range(2, KF + 1):
                dvec = dvec + acc[k]
            den = jnp.maximum(jnp.sum(dvec), jnp.float32(1e-7))
            inv = jnp.float32(1.0) / lax.broadcast_in_dim(den, (LANES,), ())
            row = p * CPASS + ci
            for k in range(KF):
                outb_v[row, pl.ds(k * LANES, LANES)] = acc[k + 1] * inv
            return 0

        lax.fori_loop(0, CPASS, fold_body, 0)
        return 0

    lax.fori_loop(0, NPASS, pass_body, 0)

    pltpu.sync_copy(outb_v, out_hbm.at[b, pl.ds(c0, C_PER_TILE), :])


@jax.jit
def kernel(inputs):
    mesh = plsc.VectorSubcoreMesh(core_axis_name="c", subcore_axis_name="s")
    run = pl.kernel(
        _body,
        out_type=jax.ShapeDtypeStruct((B, C, NBINS), jnp.float32),
        mesh=mesh,
        compiler_params=pltpu.CompilerParams(needs_layout_passes=False),
        scratch_types=[
            pltpu.VMEM((H16,), jnp.float32),
            pltpu.VMEM((HC, CPASS, W), jnp.float32),
            pltpu.VMEM((HC, CPASS, W), jnp.float32),
            pltpu.VMEM((C_PER_TILE, NBINS), jnp.float32),
            pltpu.SemaphoreType.DMA,
            pltpu.SemaphoreType.DMA,
        ],
    )
    xt = jnp.transpose(inputs, (0, 1, 3, 2))
    return jnp.transpose(run(xt), (0, 2, 1))


# speedup vs baseline: 11.2597x; 1.0200x over previous
import jax
import jax.numpy as jnp
from jax import lax
from jax.experimental import pallas as pl
from jax.experimental.pallas import tpu as pltpu
from jax.experimental.pallas import tpu_sc as plsc

B, H, W, C = 8, 224, 224, 96
NBINS = 256
NB_INT = NBINS + 1
TILES_PER_B = 4
C_PER_TILE = C // TILES_PER_B
CPASS = 8
NPASS = C_PER_TILE // CPASS
HC = 16
NCHUNK = H // HC
LANES = 16
WK = W // LANES
HROW = 273
BIN0 = 15
CROW = LANES * HROW
H16 = CPASS * CROW
KF = NBINS // LANES


def _body(x_hbm, out_hbm, h16_v, buf0, buf1, outb_v, sem0, sem1):
    cid = lax.axis_index("c")
    sid = lax.axis_index("s")
    b = cid * TILES_PER_B + sid // TILES_PER_B
    q = sid % TILES_PER_B
    c0 = q * C_PER_TILE

    zeros = jnp.zeros((LANES,), jnp.float32)
    ones = jnp.ones((LANES,), jnp.float32)
    iota = lax.iota(jnp.int32, LANES)
    lbase = iota * HROW + BIN0

    @plsc.parallel_loop(0, H16 // LANES)
    def _(i):
        h16_v[pl.ds(i * LANES, LANES)] = zeros

    def chunk_compute(buf):
        @plsc.parallel_loop(0, CPASS * HC)
        def _(i):
            ci = i >> 4
            h = i & 15
            cb = lbase + ci * CROW
            for k in range(WK):
                v = buf[h, ci, pl.ds(k * LANES, LANES)]
                idx = (v * jnp.float32(NB_INT)).astype(jnp.int32) + cb
                plsc.addupdate_scatter(h16_v, [idx], ones)

    def start(p, n, buf, sem):
        pltpu.async_copy(
            x_hbm.at[b, pl.ds(n * HC, HC), pl.ds(c0 + p * CPASS, CPASS), :],
            buf, sem)

    def wait(buf, sem):
        pltpu.make_async_copy(
            x_hbm.at[0, pl.ds(0, HC), pl.ds(0, CPASS), :], buf, sem).wait()

    def pass_body(p, _):
        start(p, 0, buf0, sem0)

        def pair_body(k, _):
            start(p, 2 * k + 1, buf1, sem1)
            wait(buf0, sem0)
            chunk_compute(buf0)

            @pl.when(k < NCHUNK // 2 - 1)
            def _():
                start(p, 2 * k + 2, buf0, sem0)

            wait(buf1, sem1)
            chunk_compute(buf1)
            return 0

        lax.fori_loop(0, NCHUNK // 2, pair_body, 0)

        def fold_body(ci, _):
            base = ci * CROW

            @plsc.parallel_loop(0, LANES, carry=tuple([zeros] * (KF + 1)))
            def acc(lane, a):
                row = base + lane * HROW
                nxt = []
                for k in range(KF + 1):
                    o = row + k * LANES
                    nxt.append(a[k] + h16_v[pl.ds(o, LANES)])
                    h16_v[pl.ds(o, LANES)] = zeros
                return tuple(nxt)

            dvec = acc[1]
            for k in range(2, KF + 1):
                dvec = dvec + acc[k]
            den = jnp.maximum(jnp.sum(dvec), jnp.float32(1e-7))
            inv = jnp.float32(1.0) / lax.broadcast_in_dim(den, (LANES,), ())
            row = p * CPASS + ci
            for k in range(KF):
                outb_v[row, pl.ds(k * LANES, LANES)] = acc[k + 1] * inv
            return 0

        lax.fori_loop(0, CPASS, fold_body, 0)
        return 0

    lax.fori_loop(0, NPASS, pass_body, 0)

    pltpu.sync_copy(outb_v, out_hbm.at[b, pl.ds(c0, C_PER_TILE), :])


@jax.jit
def kernel(inputs):
    mesh = plsc.VectorSubcoreMesh(core_axis_name="c", subcore_axis_name="s")
    run = pl.kernel(
        _body,
        out_type=jax.ShapeDtypeStruct((B, C, NBINS), jnp.float32),
        mesh=mesh,
        compiler_params=pltpu.CompilerParams(needs_layout_passes=False),
        scratch_types=[
            pltpu.VMEM((H16,), jnp.float32),
            pltpu.VMEM((HC, CPASS, W), jnp.float32),
            pltpu.VMEM((HC, CPASS, W), jnp.float32),
            pltpu.VMEM((C_PER_TILE, NBINS), jnp.float32),
            pltpu.SemaphoreType.DMA,
            pltpu.SemaphoreType.DMA,
        ],
    )
    xt = jnp.transpose(inputs, (0, 1, 3, 2))
    return jnp.transpose(run(xt), (0, 2, 1))


# speedup vs baseline: 11.3690x; 1.0097x over previous
import jax
import jax.numpy as jnp
from jax import lax
from jax.experimental import pallas as pl
from jax.experimental.pallas import tpu as pltpu
from jax.experimental.pallas import tpu_sc as plsc

B, H, W, C = 8, 224, 224, 96
NBINS = 256
NB_INT = NBINS + 1
TILES_PER_B = 4
C_PER_TILE = C // TILES_PER_B
CPASS = 8
NPASS = C_PER_TILE // CPASS
HC = 16
NCHUNK = H // HC
LANES = 16
WK = W // LANES
CSTRIDE = 258 * LANES
H16 = CPASS * CSTRIDE
NG = NBINS // LANES
SROW = 17
SG = LANES * SROW


def _body(x_hbm, out_hbm, h16_v, stg_v, buf0, buf1, outb_v, sem0, sem1):
    cid = lax.axis_index("c")
    sid = lax.axis_index("s")
    b = cid * TILES_PER_B + sid // TILES_PER_B
    q = sid % TILES_PER_B
    c0 = q * C_PER_TILE

    zeros = jnp.zeros((LANES,), jnp.float32)
    ones = jnp.ones((LANES,), jnp.float32)
    iota = lax.iota(jnp.int32, LANES)
    iota17 = iota * SROW

    @plsc.parallel_loop(0, H16 // LANES)
    def _(i):
        h16_v[pl.ds(i * LANES, LANES)] = zeros

    def chunk_compute(buf):
        @plsc.parallel_loop(0, CPASS * HC)
        def _(i):
            ci = i >> 4
            h = i & 15
            cb = iota + ci * CSTRIDE
            for k in range(WK):
                v = buf[h, ci, pl.ds(k * LANES, LANES)]
                idx = ((v * jnp.float32(NB_INT)).astype(jnp.int32) << 4) + cb
                plsc.addupdate_scatter(h16_v, [idx], ones)

    def start(p, n, buf, sem):
        pltpu.async_copy(
            x_hbm.at[b, pl.ds(n * HC, HC), pl.ds(c0 + p * CPASS, CPASS), :],
            buf, sem)

    def wait(buf, sem):
        pltpu.make_async_copy(
            x_hbm.at[0, pl.ds(0, HC), pl.ds(0, CPASS), :], buf, sem).wait()

    def pass_body(p, _):
        start(p, 0, buf0, sem0)

        def pair_body(k, _):
            start(p, 2 * k + 1, buf1, sem1)
            wait(buf0, sem0)
            chunk_compute(buf0)

            @pl.when(k < NCHUNK // 2 - 1)
            def _():
                start(p, 2 * k + 2, buf0, sem0)

            wait(buf1, sem1)
            chunk_compute(buf1)
            return 0

        lax.fori_loop(0, NCHUNK // 2, pair_body, 0)

        def fold_body(ci, _):
            cb2 = ci * CSTRIDE
            row = p * CPASS + ci

            @plsc.parallel_loop(0, NG, carry=(zeros,))
            def den_acc(g, dcar):
                sb = g * SG
                for j2 in range(LANES):
                    o = cb2 + g * (LANES * LANES) + (j2 + 1) * LANES
                    v = h16_v[pl.ds(o, LANES)]
                    plsc.store_scatter(stg_v, [iota17 + (sb + j2)], v)
                    h16_v[pl.ds(o, LANES)] = zeros
                acc = stg_v[pl.ds(sb, LANES)]
                for ll in range(1, LANES):
                    acc = acc + stg_v[pl.ds(sb + ll * SROW, LANES)]
                outb_v[row, pl.ds(g * LANES, LANES)] = acc
                return (dcar[0] + acc,)

            den = jnp.maximum(jnp.sum(den_acc[0]), jnp.float32(1e-7))
            inv = jnp.float32(1.0) / lax.broadcast_in_dim(den, (LANES,), ())
            for g in range(NG):
                outb_v[row, pl.ds(g * LANES, LANES)] = (
                    outb_v[row, pl.ds(g * LANES, LANES)] * inv)
            h16_v[pl.ds(cb2, LANES)] = zeros
            return 0

        lax.fori_loop(0, CPASS, fold_body, 0)
        return 0

    lax.fori_loop(0, NPASS, pass_body, 0)

    pltpu.sync_copy(outb_v, out_hbm.at[b, pl.ds(c0, C_PER_TILE), :])


@jax.jit
def kernel(inputs):
    mesh = plsc.VectorSubcoreMesh(core_axis_name="c", subcore_axis_name="s")
    run = pl.kernel(
        _body,
        out_type=jax.ShapeDtypeStruct((B, C, NBINS), jnp.float32),
        mesh=mesh,
        compiler_params=pltpu.CompilerParams(needs_layout_passes=False),
        scratch_types=[
            pltpu.VMEM((H16,), jnp.float32),
            pltpu.VMEM((NG * SG,), jnp.float32),
            pltpu.VMEM((HC, CPASS, W), jnp.float32),
            pltpu.VMEM((HC, CPASS, W), jnp.float32),
            pltpu.VMEM((C_PER_TILE, NBINS), jnp.float32),
            pltpu.SemaphoreType.DMA,
            pltpu.SemaphoreType.DMA,
        ],
    )
    xt = jnp.transpose(inputs, (0, 1, 3, 2))
    return jnp.transpose(run(xt), (0, 2, 1))
